# baseline (speedup 1.0000x reference)
import functools

import jax
import jax.numpy as jnp
from jax import lax
from jax.experimental import pallas as pl
from jax.experimental.pallas import tpu as pltpu
from jax.experimental.pallas import tpu_sc as plsc

E = 8
TOPK = 2
INTER = 1536
T = 2048
D = 768
NSLOTS = T * TOPK
TILE = 256
TSHIFT = 8
PADT = NSLOTS + E * TILE
NT = PADT // TILE
NWORK = 32
NTMETA = 32


def _lane_cumsum(a):
    n = a.shape[1]
    sh = 1
    while sh < n:
        z = jnp.zeros((1, sh), a.dtype)
        a = a + jnp.concatenate([z, a[:, : n - sh]], axis=1)
        sh *= 2
    return a


def _router_body(x_ref, wr_ref, pos_ref, wsl_ref, tmeta_ref, xlin_ref):
    x = x_ref[...]
    xlin_ref[...] = x
    route = lax.dot_general(
        wr_ref[...], x, (((1,), (1,)), ((), ())),
        preferred_element_type=jnp.float32)
    m0 = jnp.max(route, axis=0, keepdims=True)
    idx = lax.broadcasted_iota(jnp.int32, route.shape, 0)
    e0 = jnp.min(jnp.where(route == m0, idx, E), axis=0, keepdims=True)
    route1 = jnp.where(idx == e0, -jnp.inf, route)
    m1 = jnp.max(route1, axis=0, keepdims=True)
    e1 = jnp.min(jnp.where(route1 == m1, idx, E), axis=0, keepdims=True)
    ez = jnp.exp(m1 - m0)
    denom = 1.0 + ez
    w0 = 1.0 / denom
    w1 = ez / denom

    eslots = jnp.concatenate([e0, e1], axis=1)
    wsl_ref[...] = jnp.concatenate([w0, w1], axis=1)

    tlane = lax.broadcasted_iota(jnp.int32, (1, NTMETA), 1) * TILE
    off = jnp.zeros((1, 1), jnp.int32)
    rankpos = jnp.zeros((1, NSLOTS), jnp.int32)
    texp = jnp.zeros((1, NTMETA), jnp.int32)
    lastexp = jnp.zeros((1, 1), jnp.int32)
    for e in range(E):
        m = eslots == e
        mi = m.astype(jnp.int32)
        csum = _lane_cumsum(mi)
        cnt = csum[:, -1:]
        rankpos = jnp.where(m, csum - mi + off, rankpos)
        pc = ((cnt + (TILE - 1)) >> TSHIFT) << TSHIFT
        texp = jnp.where((tlane >= off) & (tlane < off + pc), e, texp)
        lastexp = jnp.where(cnt > 0, e, lastexp)
        off = off + pc
    tvalid = (tlane < off).astype(jnp.int32)
    texp = jnp.where(tvalid > 0, texp, lastexp)
    pos_ref[...] = rankpos
    tmeta_ref[...] = jnp.concatenate([texp, tvalid], axis=0)


def _dispatch_body(x_hbm, pos_hbm, xs_hbm, idx_v, rows_v, sem1):
    wid = lax.axis_index("s") * 2 + lax.axis_index("c")
    base = wid * (NSLOTS // NWORK)
    tok = lax.rem(base, T)
    pltpu.sync_copy(pos_hbm.at[wid], idx_v)
    pltpu.sync_copy(x_hbm.at[pl.ds(tok, NSLOTS // NWORK)], rows_v)
    pltpu.async_copy(rows_v, xs_hbm.at[idx_v], sem1).wait()


def _ffn_body(te_ref, tv_ref, xs_ref, w1_ref, b1_ref, w2_ref, b2_ref,
              ys_ref, w1bf, w2bf, prev):
    i = pl.program_id(0)

    @pl.when((i == 0) | (prev[0] != te_ref[i]))
    def _():
        w1bf[...] = w1_ref[0].astype(jnp.bfloat16)
        w2bf[...] = w2_ref[0].astype(jnp.bfloat16)
        prev[0] = te_ref[i]

    @pl.when(tv_ref[i] != 0)
    def _():
        xb = xs_ref[...].astype(jnp.bfloat16)
        h = lax.dot_general(
            xb, w1bf[...], (((1,), (1,)), ((), ())),
            preferred_element_type=jnp.float32)
        h = h + b1_ref[0]
        x0 = h[:, :INTER]
        x1 = h[:, INTER:]
        act = (x0 * (x1 * lax.logistic(x1))).astype(jnp.bfloat16)
        o = lax.dot_general(
            act, w2bf[...], (((1,), (1,)), ((), ())),
            preferred_element_type=jnp.float32)
        ys_ref[...] = o + b2_ref[0]


def _combine_body(ys_hbm, pos_hbm, wbc_hbm, out_hbm, idx0_v, idx1_v,
                  w0_v, w1_v, r0_v, r1_v, sem0, sem1):
    wid = lax.axis_index("s") * 2 + lax.axis_index("c")
    tpw = T // NWORK
    tb = wid * tpw
    pltpu.sync_copy(pos_hbm.at[pl.ds(tb, tpw)], idx0_v)
    pltpu.sync_copy(pos_hbm.at[pl.ds(T + tb, tpw)], idx1_v)
    pltpu.sync_copy(wbc_hbm.at[pl.ds(tb, tpw)], w0_v)
    pltpu.sync_copy(wbc_hbm.at[pl.ds(T + tb, tpw)], w1_v)
    c0 = pltpu.async_copy(ys_hbm.at[idx0_v], r0_v, sem0)
    c1 = pltpu.async_copy(ys_hbm.at[idx1_v], r1_v, sem1)
    c0.wait()
    c1.wait()

    def body(i, carry):
        w0 = w0_v[i, :]
        w1 = w1_v[i, :]
        for j in range(D // 16):
            sl = pl.ds(j * 16, 16)
            r0_v[i, sl] = r0_v[i, sl] * w0 + r1_v[i, sl] * w1
        return carry

    lax.fori_loop(0, tpw, body, 0)
    pltpu.sync_copy(r0_v, out_hbm.at[pl.ds(tb, tpw)])


def _copy_body(src_ref, dst_ref):
    dst_ref[...] = src_ref[...]


def _make_sc_kernels():
    mesh = plsc.VectorSubcoreMesh(core_axis_name="c", subcore_axis_name="s")
    params = pltpu.CompilerParams(use_tc_tiling_on_sc=True)
    dispatch = functools.partial(
        pl.kernel,
        out_type=jax.ShapeDtypeStruct((PADT, D), jnp.float32),
        mesh=mesh,
        compiler_params=params,
        scratch_types=[
            pltpu.VMEM((NSLOTS // NWORK,), jnp.int32),
            pltpu.VMEM((NSLOTS // NWORK, D), jnp.float32),
            pltpu.SemaphoreType.DMA,
        ],
    )(_dispatch_body)
    combine = functools.partial(
        pl.kernel,
        out_type=jax.ShapeDtypeStruct((T, D), jnp.float32),
        mesh=mesh,
        compiler_params=params,
        scratch_types=[
            pltpu.VMEM((T // NWORK,), jnp.int32),
            pltpu.VMEM((T // NWORK,), jnp.int32),
            pltpu.VMEM((T // NWORK, 16), jnp.float32),
            pltpu.VMEM((T // NWORK, 16), jnp.float32),
            pltpu.VMEM((T // NWORK, D), jnp.float32),
            pltpu.VMEM((T // NWORK, D), jnp.float32),
            pltpu.SemaphoreType.DMA,
            pltpu.SemaphoreType.DMA,
        ],
    )(_combine_body)
    return dispatch, combine


def kernel(hidden_states, Wr, W1, b1, W2, b2):
    s, b, d = hidden_states.shape
    x = hidden_states.reshape(T, D)

    pos1r, wsl1r, tmeta, xlin = pl.pallas_call(
        _router_body,
        out_shape=[
            jax.ShapeDtypeStruct((1, NSLOTS), jnp.int32),
            jax.ShapeDtypeStruct((1, NSLOTS), jnp.float32),
            jax.ShapeDtypeStruct((2, NTMETA), jnp.int32),
            jax.ShapeDtypeStruct((T, D), jnp.float32),
        ],
    )(x, Wr)

    pos32 = pos1r.reshape(NWORK, NSLOTS // NWORK)
    pos1d = pos1r.reshape(NSLOTS)
    wbc = jnp.broadcast_to(wsl1r.reshape(NSLOTS, 1), (NSLOTS, 16))
    texp = tmeta[0, :NT]
    tvalid = tmeta[1, :NT]

    dispatch, combine = _make_sc_kernels()
    xs = dispatch(xlin, pos32)

    ys = pl.pallas_call(
        _ffn_body,
        grid_spec=pltpu.PrefetchScalarGridSpec(
            num_scalar_prefetch=2,
            grid=(NT,),
            in_specs=[
                pl.BlockSpec((TILE, D), lambda i, te, tv: (i, 0)),
                pl.BlockSpec((1, 2 * INTER, D), lambda i, te, tv: (te[i], 0, 0)),
                pl.BlockSpec((1, 1, 2 * INTER), lambda i, te, tv: (te[i], 0, 0)),
                pl.BlockSpec((1, D, INTER), lambda i, te, tv: (te[i], 0, 0)),
                pl.BlockSpec((1, 1, D), lambda i, te, tv: (te[i], 0, 0)),
            ],
            out_specs=pl.BlockSpec((TILE, D), lambda i, te, tv: (i, 0)),
            scratch_shapes=[
                pltpu.VMEM((2 * INTER, D), jnp.bfloat16),
                pltpu.VMEM((D, INTER), jnp.bfloat16),
                pltpu.SMEM((1,), jnp.int32),
            ],
        ),
        out_shape=jax.ShapeDtypeStruct((PADT, D), jnp.float32),
        compiler_params=pltpu.CompilerParams(
            vmem_limit_bytes=100 * 1024 * 1024),
    )(texp, tvalid, xs, W1, b1.reshape(E, 1, 2 * INTER), W2,
      b2.reshape(E, 1, D))

    out_lin = combine(ys, pos1d, wbc)
    out = pl.pallas_call(
        _copy_body,
        out_shape=jax.ShapeDtypeStruct((T, D), jnp.float32),
    )(out_lin)
    return out.reshape(s, b, d)


# speedup vs baseline: 1.0310x; 1.0310x over previous
import functools

import jax
import jax.numpy as jnp
from jax import lax
from jax.experimental import pallas as pl
from jax.experimental.pallas import tpu as pltpu
from jax.experimental.pallas import tpu_sc as plsc

E = 8
TOPK = 2
INTER = 1536
T = 2048
D = 768
NSLOTS = T * TOPK
TILE = 256
TSHIFT = 8
PADT = NSLOTS + E * TILE
NT = PADT // TILE
NWORK = 32
NTMETA = 32


def _lane_cumsum(a):
    n = a.shape[1]
    sh = 1
    while sh < n:
        z = jnp.zeros((1, sh), a.dtype)
        a = a + jnp.concatenate([z, a[:, : n - sh]], axis=1)
        sh *= 2
    return a


def _router_body(x_ref, wr_ref, pos_ref, wsl_ref, tmeta_ref, xlin_ref):
    x = x_ref[...]
    xlin_ref[...] = x
    route = lax.dot_general(
        wr_ref[...], x, (((1,), (1,)), ((), ())),
        preferred_element_type=jnp.float32)
    m0 = jnp.max(route, axis=0, keepdims=True)
    idx = lax.broadcasted_iota(jnp.int32, route.shape, 0)
    e0 = jnp.min(jnp.where(route == m0, idx, E), axis=0, keepdims=True)
    route1 = jnp.where(idx == e0, -jnp.inf, route)
    m1 = jnp.max(route1, axis=0, keepdims=True)
    e1 = jnp.min(jnp.where(route1 == m1, idx, E), axis=0, keepdims=True)
    ez = jnp.exp(m1 - m0)
    denom = 1.0 + ez
    w0 = 1.0 / denom
    w1 = ez / denom

    eslots = jnp.concatenate([e0, e1], axis=1)
    wsl_ref[...] = jnp.concatenate([w0, w1], axis=1)

    tlane = lax.broadcasted_iota(jnp.int32, (1, NTMETA), 1) * TILE
    off = jnp.zeros((1, 1), jnp.int32)
    rankpos = jnp.zeros((1, NSLOTS), jnp.int32)
    texp = jnp.zeros((1, NTMETA), jnp.int32)
    lastexp = jnp.zeros((1, 1), jnp.int32)
    for e in range(E):
        m = eslots == e
        mi = m.astype(jnp.int32)
        csum = _lane_cumsum(mi)
        cnt = csum[:, -1:]
        rankpos = jnp.where(m, csum - mi + off, rankpos)
        pc = ((cnt + (TILE - 1)) >> TSHIFT) << TSHIFT
        texp = jnp.where((tlane >= off) & (tlane < off + pc), e, texp)
        lastexp = jnp.where(cnt > 0, e, lastexp)
        off = off + pc
    tvalid = (tlane < off).astype(jnp.int32)
    texp = jnp.where(tvalid > 0, texp, lastexp)
    pos_ref[...] = rankpos
    tmeta_ref[...] = jnp.concatenate([texp, tvalid], axis=0)


def _dispatch_body(x_hbm, pos_hbm, xs_hbm, idx_v, rows_v, sem1):
    wid = lax.axis_index("s") * 2 + lax.axis_index("c")
    base = wid * (NSLOTS // NWORK)
    tok = lax.rem(base, T)
    pltpu.sync_copy(pos_hbm.at[wid], idx_v)
    pltpu.sync_copy(x_hbm.at[pl.ds(tok, NSLOTS // NWORK)], rows_v)
    pltpu.async_copy(rows_v, xs_hbm.at[idx_v], sem1).wait()


def _ffn_body(te_ref, tv_ref, xs_ref, w1_ref, b1_ref, w2_ref, b2_ref,
              ys_ref, w1bf, w2bf, prev):
    i = pl.program_id(0)

    @pl.when((i == 0) | (prev[0] != te_ref[i]))
    def _():
        w1bf[...] = w1_ref[0].astype(jnp.bfloat16)
        w2bf[...] = w2_ref[0].astype(jnp.bfloat16)
        prev[0] = te_ref[i]

    @pl.when(tv_ref[i] != 0)
    def _():
        xb = xs_ref[...].astype(jnp.bfloat16)
        h = lax.dot_general(
            xb, w1bf[...], (((1,), (1,)), ((), ())),
            preferred_element_type=jnp.float32)
        h = h + b1_ref[0]
        x0 = h[:, :INTER]
        x1 = h[:, INTER:]
        act = (x0 * (x1 * lax.logistic(x1))).astype(jnp.bfloat16)
        o = lax.dot_general(
            act, w2bf[...], (((1,), (1,)), ((), ())),
            preferred_element_type=jnp.float32)
        ys_ref[...] = o + b2_ref[0]


def _combine_body(ys_hbm, pos_hbm, wbc_hbm, out_hbm, idx0_v, idx1_v,
                  w0_v, w1_v, r0_v, r1_v, sem0, sem1):
    wid = lax.axis_index("s") * 2 + lax.axis_index("c")
    tpw = T // NWORK
    tb = wid * tpw
    pltpu.sync_copy(pos_hbm.at[pl.ds(tb, tpw)], idx0_v)
    pltpu.sync_copy(pos_hbm.at[pl.ds(T + tb, tpw)], idx1_v)
    pltpu.sync_copy(wbc_hbm.at[pl.ds(tb, tpw)], w0_v)
    pltpu.sync_copy(wbc_hbm.at[pl.ds(T + tb, tpw)], w1_v)
    c0 = pltpu.async_copy(ys_hbm.at[idx0_v], r0_v, sem0)
    c1 = pltpu.async_copy(ys_hbm.at[idx1_v], r1_v, sem1)
    c0.wait()
    c1.wait()

    def body(i, carry):
        w0 = w0_v[i, :]
        w1 = w1_v[i, :]
        for j in range(D // 16):
            sl = pl.ds(j * 16, 16)
            r0_v[i, sl] = r0_v[i, sl] * w0 + r1_v[i, sl] * w1
        return carry

    lax.fori_loop(0, tpw, body, 0)
    pltpu.sync_copy(r0_v, out_hbm.at[pl.ds(tb, tpw)])


def _copy_body(src_ref, dst_ref):
    dst_ref[...] = src_ref[...][:, None, :]


def _make_sc_kernels():
    mesh = plsc.VectorSubcoreMesh(core_axis_name="c", subcore_axis_name="s")
    params = pltpu.CompilerParams(use_tc_tiling_on_sc=True)
    dispatch = functools.partial(
        pl.kernel,
        out_type=jax.ShapeDtypeStruct((PADT, D), jnp.float32),
        mesh=mesh,
        compiler_params=params,
        scratch_types=[
            pltpu.VMEM((NSLOTS // NWORK,), jnp.int32),
            pltpu.VMEM((NSLOTS // NWORK, D), jnp.float32),
            pltpu.SemaphoreType.DMA,
        ],
    )(_dispatch_body)
    combine = functools.partial(
        pl.kernel,
        out_type=jax.ShapeDtypeStruct((T, D), jnp.float32),
        mesh=mesh,
        compiler_params=params,
        scratch_types=[
            pltpu.VMEM((T // NWORK,), jnp.int32),
            pltpu.VMEM((T // NWORK,), jnp.int32),
            pltpu.VMEM((T // NWORK, 16), jnp.float32),
            pltpu.VMEM((T // NWORK, 16), jnp.float32),
            pltpu.VMEM((T // NWORK, D), jnp.float32),
            pltpu.VMEM((T // NWORK, D), jnp.float32),
            pltpu.SemaphoreType.DMA,
            pltpu.SemaphoreType.DMA,
        ],
    )(_combine_body)
    return dispatch, combine


def kernel(hidden_states, Wr, W1, b1, W2, b2):
    s, b, d = hidden_states.shape
    x = hidden_states.reshape(T, D)

    pos1r, wsl1r, tmeta, xlin = pl.pallas_call(
        _router_body,
        out_shape=[
            jax.ShapeDtypeStruct((1, NSLOTS), jnp.int32),
            jax.ShapeDtypeStruct((1, NSLOTS), jnp.float32),
            jax.ShapeDtypeStruct((2, NTMETA), jnp.int32),
            jax.ShapeDtypeStruct((T, D), jnp.float32),
        ],
    )(x, Wr)

    pos32 = pos1r.reshape(NWORK, NSLOTS // NWORK)
    pos1d = pos1r.reshape(NSLOTS)
    wbc = jnp.broadcast_to(wsl1r.reshape(NSLOTS, 1), (NSLOTS, 16))
    texp = tmeta[0, :NT]
    tvalid = tmeta[1, :NT]

    dispatch, combine = _make_sc_kernels()
    xs = dispatch(xlin, pos32)

    ys = pl.pallas_call(
        _ffn_body,
        grid_spec=pltpu.PrefetchScalarGridSpec(
            num_scalar_prefetch=2,
            grid=(NT,),
            in_specs=[
                pl.BlockSpec((TILE, D), lambda i, te, tv: (i, 0)),
                pl.BlockSpec((1, 2 * INTER, D), lambda i, te, tv: (te[i], 0, 0)),
                pl.BlockSpec((1, 1, 2 * INTER), lambda i, te, tv: (te[i], 0, 0)),
                pl.BlockSpec((1, D, INTER), lambda i, te, tv: (te[i], 0, 0)),
                pl.BlockSpec((1, 1, D), lambda i, te, tv: (te[i], 0, 0)),
            ],
            out_specs=pl.BlockSpec((TILE, D), lambda i, te, tv: (i, 0)),
            scratch_shapes=[
                pltpu.VMEM((2 * INTER, D), jnp.bfloat16),
                pltpu.VMEM((D, INTER), jnp.bfloat16),
                pltpu.SMEM((1,), jnp.int32),
            ],
        ),
        out_shape=jax.ShapeDtypeStruct((PADT, D), jnp.float32),
        compiler_params=pltpu.CompilerParams(
            vmem_limit_bytes=100 * 1024 * 1024),
    )(texp, tvalid, xs, W1, b1.reshape(E, 1, 2 * INTER), W2,
      b2.reshape(E, 1, D))

    out_lin = combine(ys, pos1d, wbc)
    CT = 256
    out = pl.pallas_call(
        _copy_body,
        grid=(T // CT,),
        in_specs=[pl.BlockSpec((CT, D), lambda i: (i, 0))],
        out_specs=pl.BlockSpec((CT, 1, D), lambda i: (i, 0, 0)),
        out_shape=jax.ShapeDtypeStruct((T, 1, D), jnp.float32),
    )(out_lin)
    return out


# speedup vs baseline: 1.0546x; 1.0230x over previous
import functools

import jax
import jax.numpy as jnp
from jax import lax
from jax.experimental import pallas as pl
from jax.experimental.pallas import tpu as pltpu
from jax.experimental.pallas import tpu_sc as plsc

E = 8
TOPK = 2
INTER = 1536
T = 2048
D = 768
NSLOTS = T * TOPK
TILE = 256
TSHIFT = 8
PADT = NSLOTS + E * TILE
NT = PADT // TILE
NWORK = 32
NTMETA = 32


def _lane_cumsum(a):
    n = a.shape[1]
    sh = 1
    while sh < n:
        z = jnp.zeros((1, sh), a.dtype)
        a = a + jnp.concatenate([z, a[:, : n - sh]], axis=1)
        sh *= 2
    return a


RTILE = 256


def _router_body(x_ref, wr_ref, pos_ref, wsl_ref, tmeta_ref, xlin_ref,
                 route_sc):
    i = pl.program_id(0)
    x = x_ref[...][:, 0, :]
    xlin_ref[...] = x
    route_sc[:, pl.ds(i * RTILE, RTILE)] = lax.dot_general(
        wr_ref[...], x, (((1,), (1,)), ((), ())),
        preferred_element_type=jnp.float32)

    @pl.when(i == T // RTILE - 1)
    def _():
        _router_finish(route_sc, pos_ref, wsl_ref, tmeta_ref)


def _router_finish(route_sc, pos_ref, wsl_ref, tmeta_ref):
    route = route_sc[...]
    m0 = jnp.max(route, axis=0, keepdims=True)
    idx = lax.broadcasted_iota(jnp.int32, route.shape, 0)
    e0 = jnp.min(jnp.where(route == m0, idx, E), axis=0, keepdims=True)
    route1 = jnp.where(idx == e0, -jnp.inf, route)
    m1 = jnp.max(route1, axis=0, keepdims=True)
    e1 = jnp.min(jnp.where(route1 == m1, idx, E), axis=0, keepdims=True)
    ez = jnp.exp(m1 - m0)
    denom = 1.0 + ez
    w0 = 1.0 / denom
    w1 = ez / denom

    eslots = jnp.concatenate([e0, e1], axis=1)
    wsl_ref[...] = jnp.concatenate([w0, w1], axis=1)

    tlane = lax.broadcasted_iota(jnp.int32, (1, NTMETA), 1) * TILE
    off = jnp.zeros((1, 1), jnp.int32)
    rankpos = jnp.zeros((1, NSLOTS), jnp.int32)
    texp = jnp.zeros((1, NTMETA), jnp.int32)
    lastexp = jnp.zeros((1, 1), jnp.int32)
    for e in range(E):
        m = eslots == e
        mi = m.astype(jnp.int32)
        csum = _lane_cumsum(mi)
        cnt = csum[:, -1:]
        rankpos = jnp.where(m, csum - mi + off, rankpos)
        pc = ((cnt + (TILE - 1)) >> TSHIFT) << TSHIFT
        texp = jnp.where((tlane >= off) & (tlane < off + pc), e, texp)
        lastexp = jnp.where(cnt > 0, e, lastexp)
        off = off + pc
    tvalid = (tlane < off).astype(jnp.int32)
    texp = jnp.where(tvalid > 0, texp, lastexp)
    pos_ref[...] = rankpos
    tmeta_ref[...] = jnp.concatenate([texp, tvalid], axis=0)


def _dispatch_body(x_hbm, pos_hbm, xs_hbm, idx_v, rows_v, sem1):
    wid = lax.axis_index("s") * 2 + lax.axis_index("c")
    base = wid * (NSLOTS // NWORK)
    tok = lax.rem(base, T)
    pltpu.sync_copy(pos_hbm.at[wid], idx_v)
    pltpu.sync_copy(x_hbm.at[pl.ds(tok, NSLOTS // NWORK)], rows_v)
    pltpu.async_copy(rows_v, xs_hbm.at[idx_v], sem1).wait()


def _ffn_body(te_ref, tv_ref, xs_ref, w1_ref, b1_ref, w2_ref, b2_ref,
              ys_ref, w1bf, w2bf, prev):
    i = pl.program_id(0)

    @pl.when((i == 0) | (prev[0] != te_ref[i]))
    def _():
        w1bf[...] = w1_ref[0].astype(jnp.bfloat16)
        w2bf[...] = w2_ref[0].astype(jnp.bfloat16)
        prev[0] = te_ref[i]

    @pl.when(tv_ref[i] != 0)
    def _():
        xb = xs_ref[...].astype(jnp.bfloat16)
        h = lax.dot_general(
            xb, w1bf[...], (((1,), (1,)), ((), ())),
            preferred_element_type=jnp.float32)
        h = h + b1_ref[0]
        x0 = h[:, :INTER]
        x1 = h[:, INTER:]
        act = (x0 * (x1 * lax.logistic(x1))).astype(jnp.bfloat16)
        o = lax.dot_general(
            act, w2bf[...], (((1,), (1,)), ((), ())),
            preferred_element_type=jnp.float32)
        ys_ref[...] = o + b2_ref[0]


def _combine_body(ys_hbm, pos_hbm, wbc_hbm, out_hbm, idx0_v, idx1_v,
                  w0_v, w1_v, r0_v, r1_v, sem0, sem1):
    wid = lax.axis_index("s") * 2 + lax.axis_index("c")
    tpw = T // NWORK
    tb = wid * tpw
    pltpu.sync_copy(pos_hbm.at[pl.ds(tb, tpw)], idx0_v)
    pltpu.sync_copy(pos_hbm.at[pl.ds(T + tb, tpw)], idx1_v)
    pltpu.sync_copy(wbc_hbm.at[pl.ds(tb, tpw)], w0_v)
    pltpu.sync_copy(wbc_hbm.at[pl.ds(T + tb, tpw)], w1_v)
    c0 = pltpu.async_copy(ys_hbm.at[idx0_v], r0_v, sem0)
    c1 = pltpu.async_copy(ys_hbm.at[idx1_v], r1_v, sem1)
    c0.wait()
    c1.wait()

    def body(i, carry):
        w0 = w0_v[i, :]
        w1 = w1_v[i, :]
        for j in range(D // 16):
            sl = pl.ds(j * 16, 16)
            r0_v[i, sl] = r0_v[i, sl] * w0 + r1_v[i, sl] * w1
        return carry

    lax.fori_loop(0, tpw, body, 0)
    pltpu.sync_copy(r0_v, out_hbm.at[pl.ds(tb, tpw)])


def _copy_body(src_ref, dst_ref):
    dst_ref[...] = src_ref[...][:, None, :]


def _make_sc_kernels():
    mesh = plsc.VectorSubcoreMesh(core_axis_name="c", subcore_axis_name="s")
    params = pltpu.CompilerParams(use_tc_tiling_on_sc=True)
    dispatch = functools.partial(
        pl.kernel,
        out_type=jax.ShapeDtypeStruct((PADT, D), jnp.float32),
        mesh=mesh,
        compiler_params=params,
        scratch_types=[
            pltpu.VMEM((NSLOTS // NWORK,), jnp.int32),
            pltpu.VMEM((NSLOTS // NWORK, D), jnp.float32),
            pltpu.SemaphoreType.DMA,
        ],
    )(_dispatch_body)
    combine = functools.partial(
        pl.kernel,
        out_type=jax.ShapeDtypeStruct((T, D), jnp.float32),
        mesh=mesh,
        compiler_params=params,
        scratch_types=[
            pltpu.VMEM((T // NWORK,), jnp.int32),
            pltpu.VMEM((T // NWORK,), jnp.int32),
            pltpu.VMEM((T // NWORK, 16), jnp.float32),
            pltpu.VMEM((T // NWORK, 16), jnp.float32),
            pltpu.VMEM((T // NWORK, D), jnp.float32),
            pltpu.VMEM((T // NWORK, D), jnp.float32),
            pltpu.SemaphoreType.DMA,
            pltpu.SemaphoreType.DMA,
        ],
    )(_combine_body)
    return dispatch, combine


def kernel(hidden_states, Wr, W1, b1, W2, b2):
    s, b, d = hidden_states.shape

    pos1r, wsl1r, tmeta, xlin = pl.pallas_call(
        _router_body,
        grid=(T // RTILE,),
        in_specs=[
            pl.BlockSpec((RTILE, 1, D), lambda i: (i, 0, 0)),
            pl.BlockSpec((E, D), lambda i: (0, 0)),
        ],
        out_specs=[
            pl.BlockSpec((1, NSLOTS), lambda i: (0, 0)),
            pl.BlockSpec((1, NSLOTS), lambda i: (0, 0)),
            pl.BlockSpec((2, NTMETA), lambda i: (0, 0)),
            pl.BlockSpec((RTILE, D), lambda i: (i, 0)),
        ],
        out_shape=[
            jax.ShapeDtypeStruct((1, NSLOTS), jnp.int32),
            jax.ShapeDtypeStruct((1, NSLOTS), jnp.float32),
            jax.ShapeDtypeStruct((2, NTMETA), jnp.int32),
            jax.ShapeDtypeStruct((T, D), jnp.float32),
        ],
        scratch_shapes=[pltpu.VMEM((E, T), jnp.float32)],
    )(hidden_states, Wr)

    pos32 = pos1r.reshape(NWORK, NSLOTS // NWORK)
    pos1d = pos1r.reshape(NSLOTS)
    wbc = jnp.broadcast_to(wsl1r.reshape(NSLOTS, 1), (NSLOTS, 16))
    texp = tmeta[0, :NT]
    tvalid = tmeta[1, :NT]

    dispatch, combine = _make_sc_kernels()
    xs = dispatch(xlin, pos32)

    ys = pl.pallas_call(
        _ffn_body,
        grid_spec=pltpu.PrefetchScalarGridSpec(
            num_scalar_prefetch=2,
            grid=(NT,),
            in_specs=[
                pl.BlockSpec((TILE, D), lambda i, te, tv: (i, 0)),
                pl.BlockSpec((1, 2 * INTER, D), lambda i, te, tv: (te[i], 0, 0)),
                pl.BlockSpec((1, 1, 2 * INTER), lambda i, te, tv: (te[i], 0, 0)),
                pl.BlockSpec((1, D, INTER), lambda i, te, tv: (te[i], 0, 0)),
                pl.BlockSpec((1, 1, D), lambda i, te, tv: (te[i], 0, 0)),
            ],
            out_specs=pl.BlockSpec((TILE, D), lambda i, te, tv: (i, 0)),
            scratch_shapes=[
                pltpu.VMEM((2 * INTER, D), jnp.bfloat16),
                pltpu.VMEM((D, INTER), jnp.bfloat16),
                pltpu.SMEM((1,), jnp.int32),
            ],
        ),
        out_shape=jax.ShapeDtypeStruct((PADT, D), jnp.float32),
        compiler_params=pltpu.CompilerParams(
            vmem_limit_bytes=100 * 1024 * 1024),
    )(texp, tvalid, xs, W1, b1.reshape(E, 1, 2 * INTER), W2,
      b2.reshape(E, 1, D))

    out_lin = combine(ys, pos1d, wbc)
    CT = 256
    out = pl.pallas_call(
        _copy_body,
        grid=(T // CT,),
        in_specs=[pl.BlockSpec((CT, D), lambda i: (i, 0))],
        out_specs=pl.BlockSpec((CT, 1, D), lambda i: (i, 0, 0)),
        out_shape=jax.ShapeDtypeStruct((T, 1, D), jnp.float32),
    )(out_lin)
    return out
